# Initial kernel scaffold; baseline (speedup 1.0000x reference)
import jax
import jax.numpy as jnp
from jax.experimental import pallas as pl


def kernel(inputs, Wg, A, B):
    raise NotImplementedError("write your pallas kernel here")


# speedup vs baseline: 1.0381x; 1.0381x over previous
import functools
import math

import jax
import jax.numpy as jnp
from jax import lax
from jax.experimental import pallas as pl
from jax.experimental.pallas import tpu as pltpu


def _moe_body(x_ref, wcat_ref, bflat_ref, out_ref, *, n_exp, rank, scaling):
    er = n_exp * rank
    x = x_ref[...]
    y = jnp.dot(x, wcat_ref[...], preferred_element_type=jnp.float32)
    h = y[:, :er]
    logits = y[:, er:er + n_exp]

    ei = lax.broadcasted_iota(jnp.int32, logits.shape, 1)
    m1 = jnp.max(logits, axis=-1, keepdims=True)
    i1 = jnp.min(jnp.where(logits == m1, ei, n_exp), axis=-1, keepdims=True)
    first1 = ei == i1
    masked = jnp.where(first1, -jnp.inf, logits)
    m2 = jnp.max(masked, axis=-1, keepdims=True)
    i2 = jnp.min(jnp.where(masked == m2, ei, n_exp), axis=-1, keepdims=True)
    first2 = ei == i2

    e2 = jnp.exp(m2 - m1)
    denom = 1.0 + e2
    comb = (first1.astype(jnp.float32) + first2.astype(jnp.float32) * e2) / denom

    re_ = lax.broadcasted_iota(jnp.int32, (n_exp, er), 0)
    ce_ = lax.broadcasted_iota(jnp.int32, (n_exp, er), 1)
    expand = (ce_ // rank == re_).astype(jnp.float32)
    comb64 = jnp.dot(comb, expand, preferred_element_type=jnp.float32)

    g = h * comb64
    out_ref[...] = jnp.dot(g, bflat_ref[...],
                           preferred_element_type=jnp.float32) * scaling


@functools.partial(jax.jit, static_argnames=("n_exp", "rank", "interpret"))
def _moe(x, wcat, bflat, n_exp, rank, interpret=False):
    t, d = x.shape
    out_d = bflat.shape[1]
    tb = 512
    scaling = math.sqrt(3.0 * 1.0 * d / rank)
    body = functools.partial(_moe_body, n_exp=n_exp, rank=rank, scaling=scaling)
    return pl.pallas_call(
        body,
        grid=(t // tb,),
        in_specs=[
            pl.BlockSpec((tb, d), lambda i: (i, 0)),
            pl.BlockSpec((d, 128), lambda i: (0, 0)),
            pl.BlockSpec((n_exp * rank, out_d), lambda i: (0, 0)),
        ],
        out_specs=pl.BlockSpec((tb, out_d), lambda i: (i, 0)),
        out_shape=jax.ShapeDtypeStruct((t, out_d), jnp.float32),
        compiler_params=pltpu.CompilerParams(
            dimension_semantics=("arbitrary",),
        ),
        interpret=interpret,
    )(x, wcat, bflat)


def kernel(inputs, Wg, A, B, interpret=False):
    bsz, seq, d = inputs.shape
    n_exp, rank, _ = A.shape
    out_d = B.shape[1]
    er = n_exp * rank
    x = inputs.reshape(bsz * seq, d)
    aflat = A.transpose(2, 0, 1).reshape(d, er)
    wcat = jnp.concatenate(
        [aflat, Wg.T, jnp.zeros((d, 128 - er - n_exp), jnp.float32)], axis=1)
    bflat = B.transpose(0, 2, 1).reshape(er, out_d)
    out = _moe(x, wcat, bflat, n_exp, rank, interpret=interpret)
    return out.reshape(bsz, seq, out_d)
